# Initial kernel scaffold; baseline (speedup 1.0000x reference)
import functools

import jax
import jax.numpy as jnp
from jax import lax
from jax.experimental import pallas as pl
from jax.experimental.pallas import tpu as pltpu
from jax.experimental.pallas import tpu_sc as plsc

N = 10000
E = 320000
D = 128
DH = D // 2
NC = 2
NS = 16
CH = 128
RPT = N // NS
RCH = 125
EPT = E // NS
NFULL = EPT // CH
TAIL = EPT - NFULL * CH
DEGW = 16

_mesh = plsc.VectorSubcoreMesh(core_axis_name="c", subcore_axis_name="s")


@functools.partial(
    pl.kernel,
    out_type=jax.ShapeDtypeStruct((NC, N, DEGW), jnp.float32),
    mesh=_mesh,
    scratch_types=[
        pltpu.VMEM((CH,), jnp.int32),
        pltpu.VMEM((TAIL,), jnp.int32),
        pltpu.VMEM((CH, DEGW), jnp.float32),
        pltpu.VMEM((RCH, DEGW), jnp.float32),
        pltpu.VMEM_SHARED((N, DEGW), jnp.float32),
    ],
)
def _degree_kernel(eidx_hbm, out_hbm, idx_v, idx_t, ones_v, stage_v, acc_sh):
    c = lax.axis_index("c")
    s = lax.axis_index("s")

    def init_ones(i, _):
        ones_v[i, :] = jnp.ones((DEGW,), jnp.float32)
        return 0

    lax.fori_loop(0, CH, init_ones, 0)

    def init_zero(i, _):
        stage_v[i, :] = jnp.zeros((DEGW,), jnp.float32)
        return 0

    lax.fori_loop(0, RCH, init_zero, 0)

    row0 = s * RPT
    for j in range(RPT // RCH):
        pltpu.sync_copy(stage_v, acc_sh.at[pl.ds(row0 + j * RCH, RCH)])
    plsc.subcore_barrier()

    base = s * EPT

    def chunk(g, _):
        pltpu.sync_copy(eidx_hbm.at[c].at[pl.ds(base + g * CH, CH)], idx_v)
        pltpu.sync_copy(ones_v, acc_sh.at[idx_v], add=True)
        return 0

    lax.fori_loop(0, NFULL, chunk, 0)
    pltpu.sync_copy(eidx_hbm.at[c].at[pl.ds(base + NFULL * CH, TAIL)], idx_t)
    pltpu.sync_copy(ones_v.at[pl.ds(0, TAIL)], acc_sh.at[idx_t], add=True)
    plsc.subcore_barrier()

    for j in range(RPT // RCH):
        pltpu.sync_copy(acc_sh.at[pl.ds(row0 + j * RCH, RCH)], stage_v)
        pltpu.sync_copy(stage_v, out_hbm.at[c].at[pl.ds(row0 + j * RCH, RCH)])


@functools.partial(
    pl.kernel,
    out_type=jax.ShapeDtypeStruct((NC, N, DH), jnp.float32),
    mesh=_mesh,
    scratch_types=[
        pltpu.VMEM((CH,), jnp.int32),
        pltpu.VMEM((CH,), jnp.int32),
        pltpu.VMEM((TAIL,), jnp.int32),
        pltpu.VMEM((TAIL,), jnp.int32),
        pltpu.VMEM((CH, DH), jnp.float32),
        pltpu.VMEM((TAIL, DH), jnp.float32),
        pltpu.VMEM((RCH, DH), jnp.float32),
        pltpu.VMEM_SHARED((N, DH), jnp.float32),
        pltpu.SemaphoreType.DMA,
    ],
)
def _edge_kernel(t_hbm, src_hbm, dst_hbm, out_hbm, sidx_v, didx_v, sidx_t,
                 didx_t, rows_v, rows_t, stage_v, acc_sh, sem):
    c = lax.axis_index("c")
    s = lax.axis_index("s")

    def init_zero(i, _):
        for j in range(DH // 16):
            stage_v[i, pl.ds(j * 16, 16)] = jnp.zeros((16,), jnp.float32)
        return 0

    lax.fori_loop(0, RCH, init_zero, 0)

    row0 = s * RPT
    for j in range(RPT // RCH):
        pltpu.sync_copy(stage_v, acc_sh.at[pl.ds(row0 + j * RCH, RCH)])
    plsc.subcore_barrier()

    base = s * EPT

    def chunk(g, _):
        off = base + g * CH
        pltpu.sync_copy(src_hbm.at[pl.ds(off, CH)], sidx_v)
        pltpu.sync_copy(dst_hbm.at[pl.ds(off, CH)], didx_v)
        pltpu.async_copy(t_hbm.at[c].at[sidx_v], rows_v, sem).wait()
        pltpu.sync_copy(rows_v, acc_sh.at[didx_v], add=True)
        return 0

    lax.fori_loop(0, NFULL, chunk, 0)
    off = base + NFULL * CH
    pltpu.sync_copy(src_hbm.at[pl.ds(off, TAIL)], sidx_t)
    pltpu.sync_copy(dst_hbm.at[pl.ds(off, TAIL)], didx_t)
    pltpu.async_copy(t_hbm.at[c].at[sidx_t], rows_t, sem).wait()
    pltpu.sync_copy(rows_t, acc_sh.at[didx_t], add=True)
    plsc.subcore_barrier()

    for j in range(RPT // RCH):
        pltpu.sync_copy(acc_sh.at[pl.ds(row0 + j * RCH, RCH)], stage_v)
        pltpu.sync_copy(stage_v, out_hbm.at[c].at[pl.ds(row0 + j * RCH, RCH)])


_BR = 1000


def _norm_from(deg_block):
    return lax.rsqrt(jnp.maximum(deg_block[:, 0], 1.0))


def _mm_pre_body(x_ref, deg_ref, w_ref, out_ref):
    norm_src = _norm_from(deg_ref[0])
    h = x_ref[...] * norm_src[:, None]
    y = jnp.dot(h, w_ref[...], preferred_element_type=jnp.float32)
    out_ref[0] = y[:, :DH]
    out_ref[1] = y[:, DH:]


def _mm_pre(x, deg, w):
    return pl.pallas_call(
        _mm_pre_body,
        grid=(N // _BR,),
        in_specs=[
            pl.BlockSpec((_BR, D), lambda i: (i, 0)),
            pl.BlockSpec((NC, _BR, DEGW), lambda i: (0, i, 0)),
            pl.BlockSpec((D, D), lambda i: (0, 0)),
        ],
        out_specs=pl.BlockSpec((NC, _BR, DH), lambda i: (0, i, 0)),
        out_shape=jax.ShapeDtypeStruct((NC, N, DH), jnp.float32),
    )(x, deg, w)


def _mm_mid_body(agg_ref, deg_ref, b_ref, w_ref, out_ref):
    norm_dst = _norm_from(deg_ref[1])
    norm_src = _norm_from(deg_ref[0])
    pre = jnp.concatenate([agg_ref[0], agg_ref[1]], axis=1)
    h = jnp.maximum(pre * norm_dst[:, None] + b_ref[...], 0.0)
    h = h * norm_src[:, None]
    y = jnp.dot(h, w_ref[...], preferred_element_type=jnp.float32)
    out_ref[0] = y[:, :DH]
    out_ref[1] = y[:, DH:]


def _mm_mid(agg, deg, b, w):
    return pl.pallas_call(
        _mm_mid_body,
        grid=(N // _BR,),
        in_specs=[
            pl.BlockSpec((NC, _BR, DH), lambda i: (0, i, 0)),
            pl.BlockSpec((NC, _BR, DEGW), lambda i: (0, i, 0)),
            pl.BlockSpec((1, D), lambda i: (0, 0)),
            pl.BlockSpec((D, D), lambda i: (0, 0)),
        ],
        out_specs=pl.BlockSpec((NC, _BR, DH), lambda i: (0, i, 0)),
        out_shape=jax.ShapeDtypeStruct((NC, N, DH), jnp.float32),
    )(agg, deg, b, w)


def _mm_post_body(agg_ref, deg_ref, b_ref, out_ref):
    norm_dst = _norm_from(deg_ref[1])
    pre = jnp.concatenate([agg_ref[0], agg_ref[1]], axis=1)
    out_ref[...] = jnp.maximum(pre * norm_dst[:, None] + b_ref[...], 0.0)


def _mm_post(agg, deg, b):
    return pl.pallas_call(
        _mm_post_body,
        grid=(N // _BR,),
        in_specs=[
            pl.BlockSpec((NC, _BR, DH), lambda i: (0, i, 0)),
            pl.BlockSpec((NC, _BR, DEGW), lambda i: (0, i, 0)),
            pl.BlockSpec((1, D), lambda i: (0, 0)),
        ],
        out_specs=pl.BlockSpec((_BR, D), lambda i: (i, 0)),
        out_shape=jax.ShapeDtypeStruct((N, D), jnp.float32),
    )(agg, deg, b)


def kernel(inputs, edge_index, W0, b0, W1, b1):
    src = edge_index[0].astype(jnp.int32)
    dst = edge_index[1].astype(jnp.int32)
    eidx = jnp.stack([src, dst])
    deg = _degree_kernel(eidx)
    t0 = _mm_pre(inputs, deg, W0)
    agg0 = _edge_kernel(t0, src, dst)
    t1 = _mm_mid(agg0, deg, b0.reshape(1, D), W1)
    agg1 = _edge_kernel(t1, src, dst)
    return _mm_post(agg1, deg, b1.reshape(1, D))


# speedup vs baseline: 6.3686x; 6.3686x over previous
import functools

import jax
import jax.numpy as jnp
from jax import lax
from jax.experimental import pallas as pl
from jax.experimental.pallas import tpu as pltpu
from jax.experimental.pallas import tpu_sc as plsc

N = 10000
NP = 10240
E = 320000
D = 128
NC = 2
NS = 16
CH = 128
NCHUNK = E // CH
CPC = NCHUNK // NC
RPT = NP // NS
RCH = 128

_mesh = plsc.VectorSubcoreMesh(core_axis_name="c", subcore_axis_name="s")


@functools.partial(
    pl.kernel,
    out_type=jax.ShapeDtypeStruct((NC, NS, NP), jnp.float32),
    mesh=_mesh,
    scratch_types=[
        pltpu.VMEM((CH,), jnp.int32),
        pltpu.VMEM((NP,), jnp.float32),
    ],
    compiler_params=pltpu.CompilerParams(needs_layout_passes=False),
)
def _degree_kernel(eidx_hbm, out_hbm, idx_v, hist_v):
    c = lax.axis_index("c")
    s = lax.axis_index("s")

    def init_zero(i, _):
        hist_v[pl.ds(i * 16, 16)] = jnp.zeros((16,), jnp.float32)
        return 0

    lax.fori_loop(0, NP // 16, init_zero, 0)

    nchunks = NCHUNK // NS + jnp.where(s < NCHUNK % NS, 1, 0)
    ones16 = jnp.ones((16,), jnp.float32)

    def chunk(g, _):
        off = pl.multiple_of((s + g * NS) * CH, CH)
        pltpu.sync_copy(eidx_hbm.at[c].at[pl.ds(off, CH)], idx_v)
        for j in range(CH // 16):
            idx16 = idx_v[pl.ds(j * 16, 16)]
            plsc.addupdate_scatter(hist_v, [idx16], ones16)
        return 0

    lax.fori_loop(0, nchunks, chunk, 0)
    pltpu.sync_copy(hist_v, out_hbm.at[c].at[s])


@functools.partial(
    pl.kernel,
    out_type=jax.ShapeDtypeStruct((NC, NP, D), jnp.float32),
    mesh=_mesh,
    scratch_types=[
        pltpu.VMEM((CH,), jnp.int32),
        pltpu.VMEM((CH,), jnp.int32),
        pltpu.VMEM((CH, D), jnp.float32),
        pltpu.VMEM((RCH, D), jnp.float32),
        pltpu.VMEM_SHARED((NP, D), jnp.float32),
        pltpu.SemaphoreType.DMA,
    ],
)
def _edge_kernel(t_hbm, src_hbm, dst_hbm, out_hbm, sidx_v, didx_v, rows_v,
                 stage_v, acc_sh, sem):
    c = lax.axis_index("c")
    s = lax.axis_index("s")

    def init_zero(i, _):
        for j in range(D // 16):
            stage_v[i, pl.ds(j * 16, 16)] = jnp.zeros((16,), jnp.float32)
        return 0

    lax.fori_loop(0, RCH, init_zero, 0)

    row0 = s * RPT
    for j in range(RPT // RCH):
        pltpu.sync_copy(stage_v, acc_sh.at[pl.ds(row0 + j * RCH, RCH)])
    plsc.subcore_barrier()

    nchunks = CPC // NS + jnp.where(s < CPC % NS, 1, 0)

    def chunk(g, _):
        off = pl.multiple_of((c * CPC + s + g * NS) * CH, CH)
        pltpu.sync_copy(src_hbm.at[pl.ds(off, CH)], sidx_v)
        pltpu.sync_copy(dst_hbm.at[pl.ds(off, CH)], didx_v)
        pltpu.async_copy(t_hbm.at[sidx_v], rows_v, sem).wait()
        pltpu.sync_copy(rows_v, acc_sh.at[didx_v], add=True)
        return 0

    lax.fori_loop(0, nchunks, chunk, 0)
    plsc.subcore_barrier()

    for j in range(RPT // RCH):
        pltpu.sync_copy(acc_sh.at[pl.ds(row0 + j * RCH, RCH)], stage_v)
        pltpu.sync_copy(stage_v, out_hbm.at[c].at[pl.ds(row0 + j * RCH, RCH)])


_BR = 1024


def _norm_from(deg_block):
    return lax.rsqrt(jnp.maximum(jnp.sum(deg_block, axis=0), 1.0))


def _mm_pre_body(x_ref, deg_ref, w_ref, out_ref):
    norm_src = _norm_from(deg_ref[0])
    h = x_ref[...] * norm_src[:, None]
    out_ref[...] = jnp.dot(h, w_ref[...], preferred_element_type=jnp.float32)


def _mm_pre(x, deg, w):
    return pl.pallas_call(
        _mm_pre_body,
        grid=(NP // _BR,),
        in_specs=[
            pl.BlockSpec((_BR, D), lambda i: (i, 0)),
            pl.BlockSpec((NC, NS, _BR), lambda i: (0, 0, i)),
            pl.BlockSpec((D, D), lambda i: (0, 0)),
        ],
        out_specs=pl.BlockSpec((_BR, D), lambda i: (i, 0)),
        out_shape=jax.ShapeDtypeStruct((NP, D), jnp.float32),
    )(x, deg, w)


def _mm_mid_body(agg_ref, deg_ref, b_ref, w_ref, out_ref):
    norm_dst = _norm_from(deg_ref[1])
    norm_src = _norm_from(deg_ref[0])
    pre = agg_ref[0] + agg_ref[1]
    h = jnp.maximum(pre * norm_dst[:, None] + b_ref[...], 0.0)
    h = h * norm_src[:, None]
    out_ref[...] = jnp.dot(h, w_ref[...], preferred_element_type=jnp.float32)


def _mm_mid(agg, deg, b, w):
    return pl.pallas_call(
        _mm_mid_body,
        grid=(NP // _BR,),
        in_specs=[
            pl.BlockSpec((NC, _BR, D), lambda i: (0, i, 0)),
            pl.BlockSpec((NC, NS, _BR), lambda i: (0, 0, i)),
            pl.BlockSpec((1, D), lambda i: (0, 0)),
            pl.BlockSpec((D, D), lambda i: (0, 0)),
        ],
        out_specs=pl.BlockSpec((_BR, D), lambda i: (i, 0)),
        out_shape=jax.ShapeDtypeStruct((NP, D), jnp.float32),
    )(agg, deg, b, w)


def _mm_post_body(agg_ref, deg_ref, b_ref, out_ref):
    norm_dst = _norm_from(deg_ref[1])
    pre = agg_ref[0] + agg_ref[1]
    out_ref[...] = jnp.maximum(pre * norm_dst[:, None] + b_ref[...], 0.0)


def _mm_post(agg, deg, b):
    return pl.pallas_call(
        _mm_post_body,
        grid=(NP // _BR,),
        in_specs=[
            pl.BlockSpec((NC, _BR, D), lambda i: (0, i, 0)),
            pl.BlockSpec((NC, NS, _BR), lambda i: (0, 0, i)),
            pl.BlockSpec((1, D), lambda i: (0, 0)),
        ],
        out_specs=pl.BlockSpec((_BR, D), lambda i: (i, 0)),
        out_shape=jax.ShapeDtypeStruct((NP, D), jnp.float32),
    )(agg, deg, b)


def kernel(inputs, edge_index, W0, b0, W1, b1):
    src = edge_index[0].astype(jnp.int32)
    dst = edge_index[1].astype(jnp.int32)
    eidx = jnp.stack([src, dst])
    x = jnp.pad(inputs, ((0, NP - N), (0, 0)))
    deg = _degree_kernel(eidx)
    t0 = _mm_pre(x, deg, W0)
    agg0 = _edge_kernel(t0, src, dst)
    t1 = _mm_mid(agg0, deg, b0.reshape(1, D), W1)
    agg1 = _edge_kernel(t1, src, dst)
    return _mm_post(agg1, deg, b1.reshape(1, D))[:N]
